# Initial kernel scaffold; baseline (speedup 1.0000x reference)
import jax
import jax.numpy as jnp
from jax.experimental import pallas as pl


def kernel(x, edge_index, batch, c1_w1, c1_b1, c1_g, c1_be, c1_w2, c1_b2, c2_w1, c2_b1, c2_g, c2_be, c2_w2, c2_b2, c3_w1, c3_b1, c3_g, c3_be, c3_w2, c3_b2, lin1_w, lin1_b, lin2_w, lin2_b):
    raise NotImplementedError("write your pallas kernel here")


# speedup vs baseline: 2.7376x; 2.7376x over previous
import functools

import jax
import jax.numpy as jnp
from jax import lax
from jax.experimental import pallas as pl
from jax.experimental.pallas import tpu as pltpu
from jax.experimental.pallas import tpu_sc as plsc

_N = 10000
_E = 160000
_D = 256
_HALF = 128

_NSUB = 16
_NCORE = 2
_CH = 128
_NCHUNK = -(-_E // (_NSUB * _CH))
_EP = _NSUB * _NCHUNK * _CH
_RPS = 632
_NPAD = _RPS * _NSUB

_BN_SCALE = 1.0 / (1.0 + 1e-5) ** 0.5


def _sc_body(srcs2, dstp, x2, zeros, out, sidx, didx, rows, shared, sem):
    c = lax.axis_index("c")
    s = lax.axis_index("s")
    r0 = s * _RPS
    pltpu.sync_copy(zeros.at[pl.ds(r0, _RPS)], shared.at[pl.ds(r0, _RPS)])
    plsc.subcore_barrier()

    base = c * _EP + s * _NCHUNK * _CH

    def chunk(i, carry):
        off = base + i * _CH
        doff = s * _NCHUNK * _CH + i * _CH
        pltpu.sync_copy(srcs2.at[pl.ds(off, _CH)], sidx)
        pltpu.sync_copy(dstp.at[pl.ds(doff, _CH)], didx)
        pltpu.async_copy(x2.at[sidx], rows, sem).wait()
        pltpu.sync_copy(rows, shared.at[didx], add=True)
        return carry

    lax.fori_loop(0, _NCHUNK, chunk, 0)
    plsc.subcore_barrier()
    pltpu.sync_copy(shared.at[pl.ds(r0, _RPS)],
                    out.at[pl.ds(c * _NPAD + r0, _RPS)])


@functools.partial(jax.jit, static_argnums=())
def _sc_segment_sum(srcs2, dstp, x2, zeros):
    k = pl.kernel(
        _sc_body,
        out_type=jax.ShapeDtypeStruct((_NCORE * _NPAD, _HALF), jnp.float32),
        mesh=plsc.VectorSubcoreMesh(core_axis_name="c", subcore_axis_name="s"),
        scratch_types=[
            pltpu.VMEM((_CH,), jnp.int32),
            pltpu.VMEM((_CH,), jnp.int32),
            pltpu.VMEM((_CH, _HALF), jnp.float32),
            pltpu.VMEM_SHARED((_NPAD, _HALF), jnp.float32),
            pltpu.SemaphoreType.DMA,
        ],
    )
    return k(srcs2, dstp, x2, zeros)


_BLK = 1000


def _mlp_body(x_ref, alo_ref, ahi_ref, w1_ref, b1_ref, g_ref, be_ref,
              w2_ref, b2_ref, o_ref):
    h = x_ref[...] + jnp.concatenate([alo_ref[...], ahi_ref[...]], axis=1)
    t = jnp.dot(h, w1_ref[...], preferred_element_type=jnp.float32)
    t = (t + b1_ref[...]) * (g_ref[...] * _BN_SCALE) + be_ref[...]
    t = jnp.maximum(t, 0.0)
    t = jnp.dot(t, w2_ref[...], preferred_element_type=jnp.float32)
    o_ref[...] = jnp.maximum(t + b2_ref[...], 0.0)


def _mlp(x, alo, ahi, w1, b1, g, be, w2, b2):
    grid = (_N // _BLK,)
    return pl.pallas_call(
        _mlp_body,
        grid=grid,
        in_specs=[
            pl.BlockSpec((_BLK, _D), lambda i: (i, 0)),
            pl.BlockSpec((_BLK, _HALF), lambda i: (i, 0)),
            pl.BlockSpec((_BLK, _HALF), lambda i: (i, 0)),
            pl.BlockSpec((_D, _D), lambda i: (0, 0)),
            pl.BlockSpec((1, _D), lambda i: (0, 0)),
            pl.BlockSpec((1, _D), lambda i: (0, 0)),
            pl.BlockSpec((1, _D), lambda i: (0, 0)),
            pl.BlockSpec((_D, _D), lambda i: (0, 0)),
            pl.BlockSpec((1, _D), lambda i: (0, 0)),
        ],
        out_specs=pl.BlockSpec((_BLK, _D), lambda i: (i, 0)),
        out_shape=jax.ShapeDtypeStruct((_N, _D), jnp.float32),
    )(x, alo, ahi, w1, b1.reshape(1, _D), g.reshape(1, _D),
      be.reshape(1, _D), w2, b2.reshape(1, _D))


def _head_body(h1_ref, h2_ref, h3_ref, w1_ref, b1_ref, w2_ref, b2_ref, o_ref):
    h = jnp.concatenate([h1_ref[...], h2_ref[...], h3_ref[...]], axis=1)
    t = jnp.dot(h, w1_ref[...], preferred_element_type=jnp.float32)
    t = jnp.maximum(t + b1_ref[...], 0.0)
    o = jnp.dot(t, w2_ref[...], preferred_element_type=jnp.float32)
    o = o + b2_ref[...]
    m = jnp.max(o, axis=1, keepdims=True)
    lse = jnp.log(jnp.sum(jnp.exp(o - m), axis=1, keepdims=True)) + m
    o_ref[...] = o - lse


def _head(h1, h2, h3, w1, b1, w2, b2):
    grid = (_N // _BLK,)
    dcat = 3 * _D
    dout = w2.shape[1]
    return pl.pallas_call(
        _head_body,
        grid=grid,
        in_specs=[
            pl.BlockSpec((_BLK, _D), lambda i: (i, 0)),
            pl.BlockSpec((_BLK, _D), lambda i: (i, 0)),
            pl.BlockSpec((_BLK, _D), lambda i: (i, 0)),
            pl.BlockSpec((dcat, dcat), lambda i: (0, 0)),
            pl.BlockSpec((1, dcat), lambda i: (0, 0)),
            pl.BlockSpec((dcat, dout), lambda i: (0, 0)),
            pl.BlockSpec((1, dout), lambda i: (0, 0)),
        ],
        out_specs=pl.BlockSpec((_BLK, dout), lambda i: (i, 0)),
        out_shape=jax.ShapeDtypeStruct((_N, dout), jnp.float32),
    )(h1, h2, h3, w1, b1.reshape(1, dcat), w2, b2.reshape(1, dout))


def kernel(x, edge_index, batch, c1_w1, c1_b1, c1_g, c1_be, c1_w2, c1_b2,
           c2_w1, c2_b1, c2_g, c2_be, c2_w2, c2_b2,
           c3_w1, c3_b1, c3_g, c3_be, c3_w2, c3_b2,
           lin1_w, lin1_b, lin2_w, lin2_b):
    src = edge_index[0]
    dst = edge_index[1]
    pad = _EP - _E
    src2 = 2 * src
    srcs2 = jnp.concatenate([
        src2, jnp.zeros((pad,), jnp.int32),
        src2 + 1, jnp.zeros((pad,), jnp.int32),
    ])
    dstp = jnp.concatenate([dst, jnp.full((pad,), _N, jnp.int32)])
    zeros = jnp.zeros((_NPAD, _HALF), jnp.float32)

    def gin(h, w1, b1, g, be, w2, b2):
        agg = _sc_segment_sum(srcs2, dstp, h.reshape(2 * _N, _HALF), zeros)
        alo = agg[:_N]
        ahi = agg[_NPAD:_NPAD + _N]
        return _mlp(h, alo, ahi, w1, b1, g, be, w2, b2)

    h1 = gin(x, c1_w1, c1_b1, c1_g, c1_be, c1_w2, c1_b2)
    h2 = gin(h1, c2_w1, c2_b1, c2_g, c2_be, c2_w2, c2_b2)
    h3 = gin(h2, c3_w1, c3_b1, c3_g, c3_be, c3_w2, c3_b2)
    return _head(h1, h2, h3, lin1_w, lin1_b, lin2_w, lin2_b)
